# Initial kernel scaffold; baseline (speedup 1.0000x reference)
import jax
import jax.numpy as jnp
from jax.experimental import pallas as pl


def kernel(x, edge_index, W, b, prelu_a):
    raise NotImplementedError("write your pallas kernel here")


# speedup vs baseline: 13.4210x; 13.4210x over previous
import functools

import jax
import jax.numpy as jnp
from jax import lax
from jax.experimental import pallas as pl
from jax.experimental.pallas import tpu as pltpu
from jax.experimental.pallas import tpu_sc as plsc

N = 10000
D = 128
K = 10
ALPHA = 0.2

NC = 2
NS = 16
NW = NC * NS
CH = 128
ROWS_PER_TILE = 640
NP = NS * ROWS_PER_TILE

_mesh = plsc.VectorSubcoreMesh(
    core_axis_name="c", subcore_axis_name="s", num_cores=NC, num_subcores=NS)


def _zero_vmem_rows(buf, nrows, ncolchunks):
  z = jnp.zeros((16,), jnp.float32)

  def body(i, carry):
    for k in range(ncolchunks):
      buf[i, pl.ds(16 * k, 16)] = z
    return carry

  lax.fori_loop(0, nrows, body, 0)


def _make_scatter_kernel(nch):
  @functools.partial(
      pl.kernel,
      out_type=jax.ShapeDtypeStruct((NC, NP, D), jnp.float32),
      mesh=_mesh,
      scratch_types=[
          pltpu.VMEM((nch, CH), jnp.int32),
          pltpu.VMEM((nch, CH), jnp.int32),
          pltpu.VMEM((CH, D), jnp.float32),
          pltpu.VMEM_SHARED((NP, D), jnp.float32),
          pltpu.SemaphoreType.DMA,
      ],
  )
  def scatter_kernel(u, rowi, coli, s_out, rowv, colv, buf, s_sh, sem):
    c = lax.axis_index("c")
    s = lax.axis_index("s")
    wid = NS * c + s

    _zero_vmem_rows(buf, CH, D // 16)
    for k in range(ROWS_PER_TILE // CH):
      pltpu.sync_copy(buf, s_sh.at[pl.ds(s * ROWS_PER_TILE + k * CH, CH)])
    plsc.subcore_barrier()

    pltpu.sync_copy(rowi.at[wid], rowv)
    pltpu.sync_copy(coli.at[wid], colv)

    def chunk(j, carry):
      pltpu.async_copy(u.at[rowv.at[j]], buf, sem).wait()
      pltpu.sync_copy(buf, s_sh.at[colv.at[j]], add=True)
      return carry

    lax.fori_loop(0, nch, chunk, 0)
    plsc.subcore_barrier()
    pltpu.sync_copy(s_sh.at[pl.ds(s * ROWS_PER_TILE, ROWS_PER_TILE)],
                    s_out.at[c].at[pl.ds(s * ROWS_PER_TILE, ROWS_PER_TILE)])

  return scatter_kernel


_BM = 2048


def _deg_of(dw_blk):
  return dw_blk[0, :, 0:1] + dw_blk[1, :, 0:1] + 1.0


_dw_spec = pl.BlockSpec((NC, _BM, D), lambda i: (0, i, 0))


def _prep_body(x_ref, w_ref, b_ref, dw_ref, u0_ref):
  h = jnp.dot(x_ref[...], w_ref[...],
              preferred_element_type=jnp.float32) + b_ref[...]
  u0_ref[...] = lax.rsqrt(_deg_of(dw_ref)) * h


def _combine_body(s_ref, u_ref, u0_ref, dw_ref, out_ref):
  deg = _deg_of(dw_ref)
  t = s_ref[0] + s_ref[1] + u_ref[...]
  out_ref[...] = ((1.0 - ALPHA) / deg) * t + ALPHA * u0_ref[...]


def _final_body(s_ref, u_ref, u0_ref, dw_ref, a_ref, out_ref):
  deg = _deg_of(dw_ref)
  t = s_ref[0] + s_ref[1] + u_ref[...]
  unew = ((1.0 - ALPHA) / deg) * t + ALPHA * u0_ref[...]
  z = jnp.sqrt(deg) * unew
  out_ref[...] = jnp.where(z >= 0, z, a_ref[...] * z)


_node_spec = pl.BlockSpec((_BM, D), lambda i: (i, 0))
_s_spec = pl.BlockSpec((NC, _BM, D), lambda i: (0, i, 0))
_vec_spec = pl.BlockSpec((1, D), lambda i: (0, 0))
_grid = (NP // _BM,)
_node_out = jax.ShapeDtypeStruct((NP, D), jnp.float32)

_prep_call = pl.pallas_call(
    _prep_body,
    grid=_grid,
    in_specs=[_node_spec,
              pl.BlockSpec((D, D), lambda i: (0, 0)),
              _vec_spec, _dw_spec],
    out_specs=_node_spec,
    out_shape=_node_out,
)

_combine_call = pl.pallas_call(
    _combine_body,
    grid=_grid,
    in_specs=[_s_spec, _node_spec, _node_spec, _dw_spec],
    out_specs=_node_spec,
    out_shape=_node_out,
)

_final_call = pl.pallas_call(
    _final_body,
    grid=_grid,
    in_specs=[_s_spec, _node_spec, _node_spec, _dw_spec, _vec_spec],
    out_specs=_node_spec,
    out_shape=_node_out,
)


def kernel(x, edge_index, W, b, prelu_a):
  e = edge_index.shape[1]
  nch = -(-e // (NW * CH))
  epad = NW * nch * CH
  pad = epad - e

  row = edge_index[0].astype(jnp.int32)
  col = edge_index[1].astype(jnp.int32)
  pad_ar = jnp.arange(pad, dtype=jnp.int32)
  row_p = jnp.concatenate([row, (pad_ar * 97) % N]).reshape(NW, nch, CH)
  col_p = jnp.concatenate([col, N + pad_ar % (NP - N)]).reshape(NW, nch, CH)

  xp = jnp.concatenate([x, jnp.zeros((NP - N, D), x.dtype)], axis=0)
  b2 = b.reshape(1, D)
  a2 = prelu_a.reshape(1, D)

  scat = _make_scatter_kernel(nch)
  ones = jnp.ones((NP, D), jnp.float32)
  dw = scat(ones, row_p, col_p)
  u = _prep_call(xp, W, b2, dw)
  u0 = u
  for _ in range(K - 1):
    s_parts = scat(u, row_p, col_p)
    u = _combine_call(s_parts, u, u0, dw)
  s_parts = scat(u, row_p, col_p)
  out = _final_call(s_parts, u, u0, dw, a2)
  return out[:N]


# speedup vs baseline: 17.1315x; 1.2765x over previous
import functools

import jax
import jax.numpy as jnp
from jax import lax
from jax.experimental import pallas as pl
from jax.experimental.pallas import tpu as pltpu
from jax.experimental.pallas import tpu_sc as plsc

N = 10000
D = 128
K = 10
ALPHA = 0.2

NC = 2
NS = 16
NW = NC * NS
CH = 128
ROWS_PER_TILE = 640
NP = NS * ROWS_PER_TILE

_mesh = plsc.VectorSubcoreMesh(
    core_axis_name="c", subcore_axis_name="s", num_cores=NC, num_subcores=NS)


def _zero_vmem_rows(buf, nrows, ncolchunks):
  z = jnp.zeros((16,), jnp.float32)

  def body(i, carry):
    for k in range(ncolchunks):
      buf[i, pl.ds(16 * k, 16)] = z
    return carry

  lax.fori_loop(0, nrows, body, 0)


def _make_scatter_kernel(npairs):
  halves = [npairs - npairs // 2, npairs // 2]
  nbuf = 2 * halves[0]

  @functools.partial(
      pl.kernel,
      out_type=jax.ShapeDtypeStruct((NC, NP, D), jnp.float32),
      mesh=_mesh,
      scratch_types=[
          pltpu.VMEM((nbuf, CH), jnp.int32),
          pltpu.VMEM((nbuf, CH), jnp.int32),
          pltpu.VMEM((CH, D), jnp.float32),
          pltpu.VMEM((CH, D), jnp.float32),
          pltpu.SemaphoreType.DMA,
          pltpu.SemaphoreType.DMA,
          pltpu.VMEM_SHARED((NP, D), jnp.float32),
      ],
  )
  def scatter_kernel(u, rowi, coli, s_out, rowv, colv, bufa, bufb,
                     sga, sgb, s_sh):
    c = lax.axis_index("c")
    s = lax.axis_index("s")
    wid = NS * c + s

    _zero_vmem_rows(bufa, CH, D // 16)
    for k in range(ROWS_PER_TILE // CH):
      pltpu.sync_copy(bufa, s_sh.at[pl.ds(s * ROWS_PER_TILE + k * CH, CH)])
    plsc.subcore_barrier()

    def wait_gather(buf, sem):
      pltpu.make_async_copy(u.at[rowv.at[0]], buf, sem).wait()

    cb = 0
    for h in halves:
      if h == 0:
        continue
      nload = 2 * h
      pltpu.sync_copy(rowi.at[wid].at[pl.ds(cb, nload)],
                      rowv.at[pl.ds(0, nload)])
      pltpu.sync_copy(coli.at[wid].at[pl.ds(cb, nload)],
                      colv.at[pl.ds(0, nload)])
      cb += 2 * h

      pltpu.async_copy(u.at[rowv.at[0]], bufa, sga)

      def pair(t, carry):
        j = 2 * t
        jla = jnp.where(j + 2 < nload, j + 2, 0)
        wait_gather(bufa, sga)
        pltpu.async_copy(u.at[rowv.at[j + 1]], bufb, sgb)
        pltpu.sync_copy(bufa, s_sh.at[colv.at[j]], add=True)
        wait_gather(bufb, sgb)
        pltpu.async_copy(u.at[rowv.at[jla]], bufa, sga)
        pltpu.sync_copy(bufb, s_sh.at[colv.at[j + 1]], add=True)
        return carry

      lax.fori_loop(0, h, pair, 0)
      wait_gather(bufa, sga)

    plsc.subcore_barrier()
    pltpu.sync_copy(s_sh.at[pl.ds(s * ROWS_PER_TILE, ROWS_PER_TILE)],
                    s_out.at[c].at[pl.ds(s * ROWS_PER_TILE, ROWS_PER_TILE)])

  return scatter_kernel


_BM = 2048


def _deg_of(dw_blk):
  return dw_blk[0, :, 0:1] + dw_blk[1, :, 0:1] + 1.0


_dw_spec = pl.BlockSpec((NC, _BM, D), lambda i: (0, i, 0))


def _prep_body(x_ref, w_ref, b_ref, dw_ref, u0_ref):
  h = jnp.dot(x_ref[...], w_ref[...],
              preferred_element_type=jnp.float32) + b_ref[...]
  u0_ref[...] = lax.rsqrt(_deg_of(dw_ref)) * h


def _combine_body(s_ref, u_ref, u0_ref, dw_ref, out_ref):
  deg = _deg_of(dw_ref)
  t = s_ref[0] + s_ref[1] + u_ref[...]
  out_ref[...] = ((1.0 - ALPHA) / deg) * t + ALPHA * u0_ref[...]


def _final_body(s_ref, u_ref, u0_ref, dw_ref, a_ref, out_ref):
  deg = _deg_of(dw_ref)
  t = s_ref[0] + s_ref[1] + u_ref[...]
  unew = ((1.0 - ALPHA) / deg) * t + ALPHA * u0_ref[...]
  z = jnp.sqrt(deg) * unew
  out_ref[...] = jnp.where(z >= 0, z, a_ref[...] * z)


_node_spec = pl.BlockSpec((_BM, D), lambda i: (i, 0))
_s_spec = pl.BlockSpec((NC, _BM, D), lambda i: (0, i, 0))
_vec_spec = pl.BlockSpec((1, D), lambda i: (0, 0))
_grid = (NP // _BM,)
_node_out = jax.ShapeDtypeStruct((NP, D), jnp.float32)

_prep_call = pl.pallas_call(
    _prep_body,
    grid=_grid,
    in_specs=[_node_spec,
              pl.BlockSpec((D, D), lambda i: (0, 0)),
              _vec_spec, _dw_spec],
    out_specs=_node_spec,
    out_shape=_node_out,
)

_combine_call = pl.pallas_call(
    _combine_body,
    grid=_grid,
    in_specs=[_s_spec, _node_spec, _node_spec, _dw_spec],
    out_specs=_node_spec,
    out_shape=_node_out,
)

_final_call = pl.pallas_call(
    _final_body,
    grid=_grid,
    in_specs=[_s_spec, _node_spec, _node_spec, _dw_spec, _vec_spec],
    out_specs=_node_spec,
    out_shape=_node_out,
)


def kernel(x, edge_index, W, b, prelu_a):
  e = edge_index.shape[1]
  nch = -(-e // (NW * CH))
  npairs = 8 * (-(-nch // 16))
  epad = NW * 2 * npairs * CH
  pad = epad - e

  row = edge_index[0].astype(jnp.int32)
  col = edge_index[1].astype(jnp.int32)
  pad_ar = jnp.arange(pad, dtype=jnp.int32)
  row_p = jnp.concatenate([row, (pad_ar * 97) % N]).reshape(NW, 2 * npairs, CH)
  col_p = jnp.concatenate([col, N + pad_ar % (NP - N)]).reshape(
      NW, 2 * npairs, CH)

  xp = jnp.concatenate([x, jnp.zeros((NP - N, D), x.dtype)], axis=0)
  b2 = b.reshape(1, D)
  a2 = prelu_a.reshape(1, D)

  scat = _make_scatter_kernel(npairs)
  ones = jnp.ones((NP, D), jnp.float32)
  dw = scat(ones, row_p, col_p)
  u = _prep_call(xp, W, b2, dw)
  u0 = u
  for _ in range(K - 1):
    s_parts = scat(u, row_p, col_p)
    u = _combine_call(s_parts, u, u0, dw)
  s_parts = scat(u, row_p, col_p)
  out = _final_call(s_parts, u, u0, dw, a2)
  return out[:N]


# speedup vs baseline: 19.3133x; 1.1274x over previous
import functools

import jax
import jax.numpy as jnp
from jax import lax
from jax.experimental import pallas as pl
from jax.experimental.pallas import tpu as pltpu
from jax.experimental.pallas import tpu_sc as plsc

N = 10000
D = 128
K = 10
ALPHA = 0.2

NC = 2
NS = 16
NW = NC * NS
CH = 64
NBUF = 4
ZCH = 64
ROWS_PER_TILE = 640
NP = NS * ROWS_PER_TILE

_mesh = plsc.VectorSubcoreMesh(
    core_axis_name="c", subcore_axis_name="s", num_cores=NC, num_subcores=NS)


def _zero_vmem_rows(buf, nrows, ncolchunks):
  z = jnp.zeros((16,), jnp.float32)

  def body(i, carry):
    for k in range(ncolchunks):
      buf[i, pl.ds(16 * k, 16)] = z
    return carry

  lax.fori_loop(0, nrows, body, 0)


def _make_scatter_kernel(nch):
  assert nch % (4 * 8) == 0
  halves = [nch // 4] * 4
  nbuf_idx = halves[0]

  @functools.partial(
      pl.kernel,
      out_type=jax.ShapeDtypeStruct((NC, NP, D), jnp.float32),
      mesh=_mesh,
      scratch_types=[
          pltpu.VMEM((nbuf_idx, CH), jnp.int32),
          pltpu.VMEM((nbuf_idx, CH), jnp.int32),
          [pltpu.VMEM((CH, D), jnp.float32)] * NBUF,
          [pltpu.SemaphoreType.DMA] * NBUF,
          [pltpu.SemaphoreType.DMA] * NBUF,
          pltpu.VMEM_SHARED((NP, D), jnp.float32),
      ],
  )
  def scatter_kernel(u, rowi, coli, s_out, rowv, colv, bufs, sg, ss, s_sh):
    c = lax.axis_index("c")
    s = lax.axis_index("s")
    wid = NS * c + s

    _zero_vmem_rows(bufs[0], ZCH, D // 16)
    for k in range(ROWS_PER_TILE // ZCH):
      pltpu.sync_copy(bufs[0].at[pl.ds(0, ZCH)],
                      s_sh.at[pl.ds(s * ROWS_PER_TILE + k * ZCH, ZCH)])
    plsc.subcore_barrier()

    def gather(j, b):
      pltpu.async_copy(u.at[rowv.at[j]], bufs[b], sg[b])

    def wait_gather(b):
      pltpu.make_async_copy(u.at[rowv.at[0]], bufs[b], sg[b]).wait()

    def scatter(t, b):
      pltpu.async_copy(bufs[b], s_sh.at[colv.at[t]], ss[b], add=True)

    def wait_scatter(b):
      pltpu.make_async_copy(bufs[b], s_sh.at[colv.at[0]], ss[b]).wait()

    cb = 0
    for h in halves:
      pltpu.sync_copy(rowi.at[wid].at[pl.ds(cb, h)], rowv.at[pl.ds(0, h)])
      pltpu.sync_copy(coli.at[wid].at[pl.ds(cb, h)], colv.at[pl.ds(0, h)])
      cb += h

      for b in range(NBUF - 1):
        gather(b, b)
      for i in range(NBUF):
        wait_gather(i)
        scatter(i, i)
        bn = (i + NBUF - 1) % NBUF
        if i >= 1:
          wait_scatter(bn)
        gather(i + NBUF - 1, bn)

      def group(g, carry):
        t0 = NBUF * g
        for i in range(NBUF):
          t = t0 + i
          b = i
          bn = (i + NBUF - 1) % NBUF
          wait_gather(b)
          scatter(t, b)
          wait_scatter(bn)
          jla = jnp.where(t + NBUF - 1 < h, t + NBUF - 1, 0)
          gather(jla, bn)
        return carry

      lax.fori_loop(1, h // NBUF, group, 0)
      for b in range(NBUF - 1):
        wait_gather(b % NBUF)
      wait_scatter(NBUF - 1)

    plsc.subcore_barrier()
    pltpu.sync_copy(s_sh.at[pl.ds(s * ROWS_PER_TILE, ROWS_PER_TILE)],
                    s_out.at[c].at[pl.ds(s * ROWS_PER_TILE, ROWS_PER_TILE)])

  return scatter_kernel


_BM = 2048


def _deg_of(dw_blk):
  return dw_blk[0, :, 0:1] + dw_blk[1, :, 0:1] + 1.0


_dw_spec = pl.BlockSpec((NC, _BM, D), lambda i: (0, i, 0))


def _prep_body(x_ref, w_ref, b_ref, dw_ref, u0_ref):
  h = jnp.dot(x_ref[...], w_ref[...],
              preferred_element_type=jnp.float32) + b_ref[...]
  u0_ref[...] = lax.rsqrt(_deg_of(dw_ref)) * h


def _combine_body(s_ref, u_ref, u0_ref, dw_ref, out_ref):
  deg = _deg_of(dw_ref)
  t = s_ref[0] + s_ref[1] + u_ref[...]
  out_ref[...] = ((1.0 - ALPHA) / deg) * t + ALPHA * u0_ref[...]


def _final_body(s_ref, u_ref, u0_ref, dw_ref, a_ref, out_ref):
  deg = _deg_of(dw_ref)
  t = s_ref[0] + s_ref[1] + u_ref[...]
  unew = ((1.0 - ALPHA) / deg) * t + ALPHA * u0_ref[...]
  z = jnp.sqrt(deg) * unew
  out_ref[...] = jnp.where(z >= 0, z, a_ref[...] * z)


_node_spec = pl.BlockSpec((_BM, D), lambda i: (i, 0))
_s_spec = pl.BlockSpec((NC, _BM, D), lambda i: (0, i, 0))
_vec_spec = pl.BlockSpec((1, D), lambda i: (0, 0))
_grid = (NP // _BM,)
_node_out = jax.ShapeDtypeStruct((NP, D), jnp.float32)

_prep_call = pl.pallas_call(
    _prep_body,
    grid=_grid,
    in_specs=[_node_spec,
              pl.BlockSpec((D, D), lambda i: (0, 0)),
              _vec_spec, _dw_spec],
    out_specs=_node_spec,
    out_shape=_node_out,
)

_combine_call = pl.pallas_call(
    _combine_body,
    grid=_grid,
    in_specs=[_s_spec, _node_spec, _node_spec, _dw_spec],
    out_specs=_node_spec,
    out_shape=_node_out,
)

_final_call = pl.pallas_call(
    _final_body,
    grid=_grid,
    in_specs=[_s_spec, _node_spec, _node_spec, _dw_spec, _vec_spec],
    out_specs=_node_spec,
    out_shape=_node_out,
)


def kernel(x, edge_index, W, b, prelu_a):
  e = edge_index.shape[1]
  nch = 32 * (-(-e // (NW * CH * 32)))
  epad = NW * nch * CH
  pad = epad - e

  row = edge_index[0].astype(jnp.int32)
  col = edge_index[1].astype(jnp.int32)
  pad_ar = jnp.arange(pad, dtype=jnp.int32)
  row_p = jnp.concatenate([row, (pad_ar * 97) % N]).reshape(NW, nch, CH)
  col_p = jnp.concatenate([col, N + pad_ar % (NP - N)]).reshape(NW, nch, CH)

  xp = jnp.concatenate([x, jnp.zeros((NP - N, D), x.dtype)], axis=0)
  b2 = b.reshape(1, D)
  a2 = prelu_a.reshape(1, D)

  scat = _make_scatter_kernel(nch)
  ones = jnp.ones((NP, D), jnp.float32)
  dw = scat(ones, row_p, col_p)
  u = _prep_call(xp, W, b2, dw)
  u0 = u
  for _ in range(K - 1):
    s_parts = scat(u, row_p, col_p)
    u = _combine_call(s_parts, u, u0, dw)
  s_parts = scat(u, row_p, col_p)
  out = _final_call(s_parts, u, u0, dw, a2)
  return out[:N]


# speedup vs baseline: 20.1385x; 1.0427x over previous
import functools

import jax
import jax.numpy as jnp
from jax import lax
from jax.experimental import pallas as pl
from jax.experimental.pallas import tpu as pltpu
from jax.experimental.pallas import tpu_sc as plsc

N = 10000
D = 128
K = 10
ALPHA = 0.2

NC = 2
NS = 16
NW = NC * NS
CH = 64
NBUF = 4
ZCH = 64
ROWS_PER_TILE = 640
NP = NS * ROWS_PER_TILE

_mesh = plsc.VectorSubcoreMesh(
    core_axis_name="c", subcore_axis_name="s", num_cores=NC, num_subcores=NS)


def _zero_vmem_rows(buf, nrows, ncolchunks):
  z = jnp.zeros((16,), jnp.float32)

  def body(i, carry):
    for k in range(ncolchunks):
      buf[i, pl.ds(16 * k, 16)] = z
    return carry

  lax.fori_loop(0, nrows, body, 0)


def _make_scatter_kernel(nch):
  assert nch % (4 * 8) == 0
  halves = [nch // 4] * 4
  nbuf_idx = halves[0]

  @functools.partial(
      pl.kernel,
      out_type=jax.ShapeDtypeStruct((NC, NP, D), jnp.float32),
      mesh=_mesh,
      scratch_types=[
          pltpu.VMEM((nbuf_idx, CH), jnp.int32),
          pltpu.VMEM((nbuf_idx, CH), jnp.int32),
          [pltpu.VMEM((CH, D), jnp.float32)] * NBUF,
          [pltpu.SemaphoreType.DMA] * NBUF,
          [pltpu.SemaphoreType.DMA] * NBUF,
          pltpu.VMEM_SHARED((NP, D), jnp.float32),
      ],
  )
  def scatter_kernel(u, rowi, coli, s_out, rowv, colv, bufs, sg, ss, s_sh):
    c = lax.axis_index("c")
    s = lax.axis_index("s")
    wid = NS * c + s

    _zero_vmem_rows(bufs[0], ZCH, D // 16)
    for k in range(ROWS_PER_TILE // ZCH):
      pltpu.sync_copy(bufs[0].at[pl.ds(0, ZCH)],
                      s_sh.at[pl.ds(s * ROWS_PER_TILE + k * ZCH, ZCH)])
    plsc.subcore_barrier()

    def gather(j, b):
      pltpu.async_copy(u.at[rowv.at[j]], bufs[b], sg[b])

    def wait_gather(b):
      pltpu.make_async_copy(u.at[rowv.at[0]], bufs[b], sg[b]).wait()

    def scatter(t, b):
      pltpu.async_copy(bufs[b], s_sh.at[colv.at[t]], ss[b], add=True)

    def wait_scatter(b):
      pltpu.make_async_copy(bufs[b], s_sh.at[colv.at[0]], ss[b]).wait()

    cb = 0
    for h in halves:
      pltpu.sync_copy(rowi.at[wid].at[pl.ds(cb, h)], rowv.at[pl.ds(0, h)])
      pltpu.sync_copy(coli.at[wid].at[pl.ds(cb, h)], colv.at[pl.ds(0, h)])
      cb += h

      for b in range(NBUF - 1):
        gather(b, b)
      for i in range(NBUF):
        wait_gather(i)
        scatter(i, i)
        bn = (i + NBUF - 1) % NBUF
        if i >= 1:
          wait_scatter(bn)
        gather(i + NBUF - 1, bn)

      def group(g, carry):
        t0 = NBUF * g
        for i in range(NBUF):
          t = t0 + i
          b = i
          bn = (i + NBUF - 1) % NBUF
          wait_gather(b)
          scatter(t, b)
          wait_scatter(bn)
          jla = jnp.where(t + NBUF - 1 < h, t + NBUF - 1, 0)
          gather(jla, bn)
        return carry

      lax.fori_loop(1, h // NBUF, group, 0)
      for b in range(NBUF - 1):
        wait_gather(b % NBUF)
      wait_scatter(NBUF - 1)

    plsc.subcore_barrier()
    pltpu.sync_copy(s_sh.at[pl.ds(s * ROWS_PER_TILE, ROWS_PER_TILE)],
                    s_out.at[c].at[pl.ds(s * ROWS_PER_TILE, ROWS_PER_TILE)])

  return scatter_kernel


def _make_deg_kernel(nch):
  @functools.partial(
      pl.kernel,
      out_type=jax.ShapeDtypeStruct((NC, NP, D), jnp.float32),
      mesh=_mesh,
      scratch_types=[
          pltpu.VMEM((nch, CH), jnp.int32),
          pltpu.VMEM((CH, D), jnp.float32),
          [pltpu.SemaphoreType.DMA] * NBUF,
          pltpu.VMEM_SHARED((NP, D), jnp.float32),
      ],
  )
  def deg_kernel(coli, dw, colv, obuf, ss, s_sh):
    c = lax.axis_index("c")
    s = lax.axis_index("s")
    wid = NS * c + s

    _zero_vmem_rows(obuf, ZCH, D // 16)
    for k in range(ROWS_PER_TILE // ZCH):
      pltpu.sync_copy(obuf.at[pl.ds(0, ZCH)],
                      s_sh.at[pl.ds(s * ROWS_PER_TILE + k * ZCH, ZCH)])
    one = jnp.ones((16,), jnp.float32)

    def fill(i, carry):
      for k in range(D // 16):
        obuf[i, pl.ds(16 * k, 16)] = one
      return carry

    lax.fori_loop(0, CH, fill, 0)
    pltpu.sync_copy(coli.at[wid], colv)
    plsc.subcore_barrier()

    def scatter(t, b):
      pltpu.async_copy(obuf, s_sh.at[colv.at[t]], ss[b], add=True)

    def wait_scatter(b):
      pltpu.make_async_copy(obuf, s_sh.at[colv.at[0]], ss[b]).wait()

    for b in range(NBUF):
      scatter(b, b)

    def group(g, carry):
      t0 = NBUF * g
      for i in range(NBUF):
        wait_scatter(i)
        scatter(t0 + i, i)
      return carry

    lax.fori_loop(1, nch // NBUF, group, 0)
    for b in range(NBUF):
      wait_scatter(b)
    plsc.subcore_barrier()
    pltpu.sync_copy(s_sh.at[pl.ds(s * ROWS_PER_TILE, ROWS_PER_TILE)],
                    dw.at[c].at[pl.ds(s * ROWS_PER_TILE, ROWS_PER_TILE)])

  return deg_kernel


_BM = 2048


_dw_spec = pl.BlockSpec((NC, _BM, D), lambda i: (0, i, 0))
_degc_spec = pl.BlockSpec((_BM, 16), lambda i: (i, 0))


def _prep_body(x_ref, w_ref, b_ref, dw_ref, u0_ref, degc_ref):
  h = jnp.dot(x_ref[...], w_ref[...],
              preferred_element_type=jnp.float32) + b_ref[...]
  deg = dw_ref[0, :, 0:1] + dw_ref[1, :, 0:1] + 1.0
  u0_ref[...] = lax.rsqrt(deg) * h
  degc_ref[...] = jnp.broadcast_to(deg, (deg.shape[0], 16))


def _combine_body(s_ref, u_ref, u0_ref, degc_ref, out_ref):
  deg = degc_ref[:, 0:1]
  t = s_ref[0] + s_ref[1] + u_ref[...]
  out_ref[...] = ((1.0 - ALPHA) / deg) * t + ALPHA * u0_ref[...]


def _final_body(s_ref, u_ref, u0_ref, degc_ref, a_ref, out_ref):
  deg = degc_ref[:, 0:1]
  t = s_ref[0] + s_ref[1] + u_ref[...]
  unew = ((1.0 - ALPHA) / deg) * t + ALPHA * u0_ref[...]
  z = jnp.sqrt(deg) * unew
  out_ref[...] = jnp.where(z >= 0, z, a_ref[...] * z)


_node_spec = pl.BlockSpec((_BM, D), lambda i: (i, 0))
_s_spec = pl.BlockSpec((NC, _BM, D), lambda i: (0, i, 0))
_vec_spec = pl.BlockSpec((1, D), lambda i: (0, 0))
_grid = (NP // _BM,)
_node_out = jax.ShapeDtypeStruct((NP, D), jnp.float32)

_prep_call = pl.pallas_call(
    _prep_body,
    grid=_grid,
    in_specs=[_node_spec,
              pl.BlockSpec((D, D), lambda i: (0, 0)),
              _vec_spec, _dw_spec],
    out_specs=[_node_spec, _degc_spec],
    out_shape=[_node_out, jax.ShapeDtypeStruct((NP, 16), jnp.float32)],
)

_combine_call = pl.pallas_call(
    _combine_body,
    grid=_grid,
    in_specs=[_s_spec, _node_spec, _node_spec, _degc_spec],
    out_specs=_node_spec,
    out_shape=_node_out,
)

_final_call = pl.pallas_call(
    _final_body,
    grid=_grid,
    in_specs=[_s_spec, _node_spec, _node_spec, _degc_spec, _vec_spec],
    out_specs=_node_spec,
    out_shape=_node_out,
)


def kernel(x, edge_index, W, b, prelu_a):
  e = edge_index.shape[1]
  nch = 32 * (-(-e // (NW * CH * 32)))
  epad = NW * nch * CH
  pad = epad - e

  row = edge_index[0].astype(jnp.int32)
  col = edge_index[1].astype(jnp.int32)
  pad_ar = jnp.arange(pad, dtype=jnp.int32)
  row_p = jnp.concatenate([row, (pad_ar * 97) % N]).reshape(NW, nch, CH)
  col_p = jnp.concatenate([col, N + pad_ar % (NP - N)]).reshape(NW, nch, CH)

  xp = jnp.concatenate([x, jnp.zeros((NP - N, D), x.dtype)], axis=0)
  b2 = b.reshape(1, D)
  a2 = prelu_a.reshape(1, D)

  scat = _make_scatter_kernel(nch)
  dw = _make_deg_kernel(nch)(col_p)
  u, degc = _prep_call(xp, W, b2, dw)
  u0 = u
  for _ in range(K - 1):
    s_parts = scat(u, row_p, col_p)
    u = _combine_call(s_parts, u, u0, degc)
  s_parts = scat(u, row_p, col_p)
  out = _final_call(s_parts, u, u0, degc, a2)
  return out[:N]


# speedup vs baseline: 20.4250x; 1.0142x over previous
import functools

import jax
import jax.numpy as jnp
from jax import lax
from jax.experimental import pallas as pl
from jax.experimental.pallas import tpu as pltpu
from jax.experimental.pallas import tpu_sc as plsc

N = 10000
D = 128
K = 10
ALPHA = 0.2

NC = 2
NS = 16
NW = NC * NS
CH = 64
NBUF = 4
ZCH = 64
ROWS_PER_TILE = 640
NP = NS * ROWS_PER_TILE

_mesh = plsc.VectorSubcoreMesh(
    core_axis_name="c", subcore_axis_name="s", num_cores=NC, num_subcores=NS)


def _zero_vmem_rows(buf, nrows, ncolchunks):
  z = jnp.zeros((16,), jnp.float32)

  def body(i, carry):
    for k in range(ncolchunks):
      buf[i, pl.ds(16 * k, 16)] = z
    return carry

  lax.fori_loop(0, nrows, body, 0)


def _make_scatter_kernel(nch):
  assert nch % (4 * 8) == 0
  halves = [nch // 4] * 4
  nbuf_idx = halves[0]

  @functools.partial(
      pl.kernel,
      out_type=jax.ShapeDtypeStruct((NC, NP, D), jnp.float32),
      mesh=_mesh,
      scratch_types=[
          pltpu.VMEM((nbuf_idx, CH), jnp.int32),
          pltpu.VMEM((nbuf_idx, CH), jnp.int32),
          [pltpu.VMEM((CH, D), jnp.float32)] * NBUF,
          [pltpu.SemaphoreType.DMA] * NBUF,
          [pltpu.SemaphoreType.DMA] * NBUF,
          pltpu.VMEM_SHARED((NP, D), jnp.float32),
      ],
  )
  def scatter_kernel(u, rowi, coli, s_out, rowv, colv, bufs, sg, ss, s_sh):
    c = lax.axis_index("c")
    s = lax.axis_index("s")
    wid = NS * c + s

    _zero_vmem_rows(bufs[0], ZCH, D // 16)
    idx_a = pltpu.async_copy(rowi.at[wid].at[pl.ds(0, halves[0])],
                             rowv.at[pl.ds(0, halves[0])], sg[0])
    idx_b = pltpu.async_copy(coli.at[wid].at[pl.ds(0, halves[0])],
                             colv.at[pl.ds(0, halves[0])], sg[1])
    nz = ROWS_PER_TILE // ZCH
    for k in range(nz):
      b = k % NBUF
      if k >= NBUF:
        pltpu.make_async_copy(
            bufs[0], s_sh.at[pl.ds(0, ZCH)], ss[b]).wait()
      pltpu.async_copy(bufs[0],
                       s_sh.at[pl.ds(s * ROWS_PER_TILE + k * ZCH, ZCH)],
                       ss[b])
    for b in range(min(NBUF, nz)):
      pltpu.make_async_copy(bufs[0], s_sh.at[pl.ds(0, ZCH)], ss[b]).wait()
    idx_a.wait()
    idx_b.wait()
    plsc.subcore_barrier()

    def gather(j, b):
      pltpu.async_copy(u.at[rowv.at[j]], bufs[b], sg[b])

    def wait_gather(b):
      pltpu.make_async_copy(u.at[rowv.at[0]], bufs[b], sg[b]).wait()

    def scatter(t, b):
      pltpu.async_copy(bufs[b], s_sh.at[colv.at[t]], ss[b], add=True)

    def wait_scatter(b):
      pltpu.make_async_copy(bufs[b], s_sh.at[colv.at[0]], ss[b]).wait()

    cb = 0
    for p, h in enumerate(halves):
      if p > 0:
        pltpu.sync_copy(rowi.at[wid].at[pl.ds(cb, h)], rowv.at[pl.ds(0, h)])
        pltpu.sync_copy(coli.at[wid].at[pl.ds(cb, h)], colv.at[pl.ds(0, h)])
      cb += h

      for b in range(NBUF - 1):
        gather(b, b)
      for i in range(NBUF):
        wait_gather(i)
        scatter(i, i)
        bn = (i + NBUF - 1) % NBUF
        if i >= 1:
          wait_scatter(bn)
        gather(i + NBUF - 1, bn)

      def group(g, carry):
        t0 = NBUF * g
        for i in range(NBUF):
          t = t0 + i
          b = i
          bn = (i + NBUF - 1) % NBUF
          wait_gather(b)
          scatter(t, b)
          wait_scatter(bn)
          jla = jnp.where(t + NBUF - 1 < h, t + NBUF - 1, 0)
          gather(jla, bn)
        return carry

      lax.fori_loop(1, h // NBUF, group, 0)
      for b in range(NBUF - 1):
        wait_gather(b % NBUF)
      wait_scatter(NBUF - 1)

    plsc.subcore_barrier()
    pltpu.sync_copy(s_sh.at[pl.ds(s * ROWS_PER_TILE, ROWS_PER_TILE)],
                    s_out.at[c].at[pl.ds(s * ROWS_PER_TILE, ROWS_PER_TILE)])

  return scatter_kernel


def _make_deg_kernel(nch):
  @functools.partial(
      pl.kernel,
      out_type=jax.ShapeDtypeStruct((NC, NP, D), jnp.float32),
      mesh=_mesh,
      scratch_types=[
          pltpu.VMEM((nch, CH), jnp.int32),
          pltpu.VMEM((CH, D), jnp.float32),
          [pltpu.SemaphoreType.DMA] * NBUF,
          pltpu.VMEM_SHARED((NP, D), jnp.float32),
      ],
  )
  def deg_kernel(coli, dw, colv, obuf, ss, s_sh):
    c = lax.axis_index("c")
    s = lax.axis_index("s")
    wid = NS * c + s

    _zero_vmem_rows(obuf, ZCH, D // 16)
    nz = ROWS_PER_TILE // ZCH
    for k in range(nz):
      b = k % NBUF
      if k >= NBUF:
        pltpu.make_async_copy(obuf, s_sh.at[pl.ds(0, ZCH)], ss[b]).wait()
      pltpu.async_copy(obuf, s_sh.at[pl.ds(s * ROWS_PER_TILE + k * ZCH, ZCH)],
                       ss[b])
    pltpu.sync_copy(coli.at[wid], colv)
    for b in range(min(NBUF, nz)):
      pltpu.make_async_copy(obuf, s_sh.at[pl.ds(0, ZCH)], ss[b]).wait()
    one = jnp.ones((16,), jnp.float32)

    def fill(i, carry):
      for k in range(D // 16):
        obuf[i, pl.ds(16 * k, 16)] = one
      return carry

    lax.fori_loop(0, CH, fill, 0)
    plsc.subcore_barrier()

    def scatter(t, b):
      pltpu.async_copy(obuf, s_sh.at[colv.at[t]], ss[b], add=True)

    def wait_scatter(b):
      pltpu.make_async_copy(obuf, s_sh.at[colv.at[0]], ss[b]).wait()

    for b in range(NBUF):
      scatter(b, b)

    def group(g, carry):
      t0 = NBUF * g
      for i in range(NBUF):
        wait_scatter(i)
        scatter(t0 + i, i)
      return carry

    lax.fori_loop(1, nch // NBUF, group, 0)
    for b in range(NBUF):
      wait_scatter(b)
    plsc.subcore_barrier()
    pltpu.sync_copy(s_sh.at[pl.ds(s * ROWS_PER_TILE, ROWS_PER_TILE)],
                    dw.at[c].at[pl.ds(s * ROWS_PER_TILE, ROWS_PER_TILE)])

  return deg_kernel


_BM = 2048


_dw_spec = pl.BlockSpec((NC, _BM, D), lambda i: (0, i, 0))
_degc_spec = pl.BlockSpec((_BM, 16), lambda i: (i, 0))


def _prep_body(x_ref, w_ref, b_ref, dw_ref, u0_ref, degc_ref):
  h = jnp.dot(x_ref[...], w_ref[...],
              preferred_element_type=jnp.float32) + b_ref[...]
  deg = dw_ref[0, :, 0:1] + dw_ref[1, :, 0:1] + 1.0
  u0_ref[...] = lax.rsqrt(deg) * h
  degc_ref[...] = jnp.broadcast_to(deg, (deg.shape[0], 16))


def _combine_body(s_ref, u_ref, u0_ref, degc_ref, out_ref):
  deg = degc_ref[:, 0:1]
  t = s_ref[0] + s_ref[1] + u_ref[...]
  out_ref[...] = ((1.0 - ALPHA) / deg) * t + ALPHA * u0_ref[...]


def _final_body(s_ref, u_ref, u0_ref, degc_ref, a_ref, out_ref):
  deg = degc_ref[:, 0:1]
  t = s_ref[0] + s_ref[1] + u_ref[...]
  unew = ((1.0 - ALPHA) / deg) * t + ALPHA * u0_ref[...]
  z = jnp.sqrt(deg) * unew
  out_ref[...] = jnp.where(z >= 0, z, a_ref[...] * z)


_node_spec = pl.BlockSpec((_BM, D), lambda i: (i, 0))
_s_spec = pl.BlockSpec((NC, _BM, D), lambda i: (0, i, 0))
_vec_spec = pl.BlockSpec((1, D), lambda i: (0, 0))
_grid = (NP // _BM,)
_node_out = jax.ShapeDtypeStruct((NP, D), jnp.float32)

_prep_call = pl.pallas_call(
    _prep_body,
    grid=_grid,
    in_specs=[_node_spec,
              pl.BlockSpec((D, D), lambda i: (0, 0)),
              _vec_spec, _dw_spec],
    out_specs=[_node_spec, _degc_spec],
    out_shape=[_node_out, jax.ShapeDtypeStruct((NP, 16), jnp.float32)],
)

_combine_call = pl.pallas_call(
    _combine_body,
    grid=_grid,
    in_specs=[_s_spec, _node_spec, _node_spec, _degc_spec],
    out_specs=_node_spec,
    out_shape=_node_out,
)

_final_call = pl.pallas_call(
    _final_body,
    grid=_grid,
    in_specs=[_s_spec, _node_spec, _node_spec, _degc_spec, _vec_spec],
    out_specs=_node_spec,
    out_shape=_node_out,
)


def kernel(x, edge_index, W, b, prelu_a):
  e = edge_index.shape[1]
  nch = 32 * (-(-e // (NW * CH * 32)))
  epad = NW * nch * CH
  pad = epad - e

  row = edge_index[0].astype(jnp.int32)
  col = edge_index[1].astype(jnp.int32)
  pad_ar = jnp.arange(pad, dtype=jnp.int32)
  row_p = jnp.concatenate([row, (pad_ar * 97) % N]).reshape(NW, nch, CH)
  col_p = jnp.concatenate([col, N + pad_ar % (NP - N)]).reshape(NW, nch, CH)

  xp = jnp.concatenate([x, jnp.zeros((NP - N, D), x.dtype)], axis=0)
  b2 = b.reshape(1, D)
  a2 = prelu_a.reshape(1, D)

  scat = _make_scatter_kernel(nch)
  dw = _make_deg_kernel(nch)(col_p)
  u, degc = _prep_call(xp, W, b2, dw)
  u0 = u
  for _ in range(K - 1):
    s_parts = scat(u, row_p, col_p)
    u = _combine_call(s_parts, u, u0, degc)
  s_parts = scat(u, row_p, col_p)
  out = _final_call(s_parts, u, u0, degc, a2)
  return out[:N]
